# baseline (speedup 1.0000x reference)
import functools

import jax
import jax.numpy as jnp
from jax import lax
from jax.experimental import pallas as pl
from jax.experimental.pallas import tpu as pltpu
from jax.experimental.pallas import tpu_sc as plsc

_B = 4096
_D = 64


def _sc_gather(W, U, H, I):
    info = plsc.get_sparse_core_info()
    nc, ns = info.num_cores, info.num_subcores
    nw = nc * ns
    bpw = _B // nw
    mesh = plsc.VectorSubcoreMesh(core_axis_name="c", subcore_axis_name="s")

    @functools.partial(
        pl.kernel,
        mesh=mesh,
        compiler_params=pltpu.CompilerParams(use_tc_tiling_on_sc=False),
        out_type=[
            jax.ShapeDtypeStruct((_B, _D), jnp.float32),
            jax.ShapeDtypeStruct((_B, _D), jnp.float32),
        ],
        scratch_types=[
            pltpu.VMEM((bpw,), jnp.int32),
            pltpu.VMEM((bpw, _D), jnp.float32),
            pltpu.VMEM((bpw,), jnp.int32),
            pltpu.VMEM((bpw, _D), jnp.float32),
            pltpu.SemaphoreType.DMA,
            pltpu.SemaphoreType.DMA,
        ],
    )
    def gather_k(W_hbm, U_hbm, H_hbm, I_hbm, wout, hout,
                 uidx, wrows, iidx, hrows, s1, s2):
        wid = lax.axis_index("s") * nc + lax.axis_index("c")
        base = wid * bpw
        pltpu.sync_copy(U_hbm.at[pl.ds(base, bpw)], uidx)
        pltpu.sync_copy(I_hbm.at[pl.ds(base, bpw)], iidx)
        c1 = pltpu.async_copy(W_hbm.at[uidx], wrows, s1)
        c2 = pltpu.async_copy(H_hbm.at[iidx], hrows, s2)
        c1.wait()
        c2.wait()
        pltpu.sync_copy(wrows, wout.at[pl.ds(base, bpw)])
        pltpu.sync_copy(hrows, hout.at[pl.ds(base, bpw)])

    return gather_k(W, U, H, I)


def _tc_matmul(wu, hi):
    bm = 512

    def mm(w_ref, h_ref, o_ref):
        o_ref[...] = lax.dot_general(
            w_ref[...], h_ref[...],
            (((1,), (1,)), ((), ())),
            preferred_element_type=jnp.float32,
        )

    return pl.pallas_call(
        mm,
        grid=(_B // bm,),
        in_specs=[
            pl.BlockSpec((bm, _D), lambda i: (i, 0)),
            pl.BlockSpec((_B, _D), lambda i: (0, 0)),
        ],
        out_specs=pl.BlockSpec((bm, _B), lambda i: (i, 0)),
        out_shape=jax.ShapeDtypeStruct((_B, _B), jnp.float32),
    )(wu, hi)


def kernel(U, I, W, H):
    wu, hi = _sc_gather(W, U, H, I)
    return _tc_matmul(wu, hi)


# speedup vs baseline: 1.0020x; 1.0020x over previous
import functools

import jax
import jax.numpy as jnp
from jax import lax
from jax.experimental import pallas as pl
from jax.experimental.pallas import tpu as pltpu
from jax.experimental.pallas import tpu_sc as plsc

_B = 4096
_D = 64


def _sc_gather(W, U, H, I):
    info = plsc.get_sparse_core_info()
    nc, ns = info.num_cores, info.num_subcores
    nw = nc * ns
    bpw = _B // nw
    mesh = plsc.VectorSubcoreMesh(core_axis_name="c", subcore_axis_name="s")

    @functools.partial(
        pl.kernel,
        mesh=mesh,
        compiler_params=pltpu.CompilerParams(use_tc_tiling_on_sc=False),
        out_type=[
            jax.ShapeDtypeStruct((_B, _D), jnp.float32),
            jax.ShapeDtypeStruct((_B, _D), jnp.float32),
        ],
        scratch_types=[
            pltpu.VMEM((bpw,), jnp.int32),
            pltpu.VMEM((bpw, _D), jnp.float32),
            pltpu.VMEM((bpw,), jnp.int32),
            pltpu.VMEM((bpw, _D), jnp.float32),
            pltpu.SemaphoreType.DMA,
            pltpu.SemaphoreType.DMA,
        ],
    )
    def gather_k(W_hbm, U_hbm, H_hbm, I_hbm, wout, hout,
                 uidx, wrows, iidx, hrows, s1, s2):
        wid = lax.axis_index("s") * nc + lax.axis_index("c")
        base = wid * bpw
        pltpu.sync_copy(U_hbm.at[pl.ds(base, bpw)], uidx)
        pltpu.sync_copy(I_hbm.at[pl.ds(base, bpw)], iidx)
        c1 = pltpu.async_copy(W_hbm.at[uidx], wrows, s1)
        c2 = pltpu.async_copy(H_hbm.at[iidx], hrows, s2)
        c1.wait()
        c2.wait()
        pltpu.sync_copy(wrows, wout.at[pl.ds(base, bpw)])
        pltpu.sync_copy(hrows, hout.at[pl.ds(base, bpw)])

    return gather_k(W, U, H, I)


def _tc_matmul(wu, hi):
    bm = 512

    def mm(w_ref, h_ref, o_ref):
        o_ref[...] = lax.dot_general(
            w_ref[...].astype(jnp.bfloat16), h_ref[...].astype(jnp.bfloat16),
            (((1,), (1,)), ((), ())),
            preferred_element_type=jnp.float32,
        )

    return pl.pallas_call(
        mm,
        grid=(_B // bm,),
        in_specs=[
            pl.BlockSpec((bm, _D), lambda i: (i, 0)),
            pl.BlockSpec((_B, _D), lambda i: (0, 0)),
        ],
        out_specs=pl.BlockSpec((bm, _B), lambda i: (i, 0)),
        out_shape=jax.ShapeDtypeStruct((_B, _B), jnp.float32),
    )(wu, hi)


def kernel(U, I, W, H):
    wu, hi = _sc_gather(W, U, H, I)
    return _tc_matmul(wu, hi)


# speedup vs baseline: 1.5743x; 1.5711x over previous
import functools

import jax
import jax.numpy as jnp
from jax import lax
from jax.experimental import pallas as pl
from jax.experimental.pallas import tpu as pltpu
from jax.experimental.pallas import tpu_sc as plsc

_B = 4096
_D = 64


def _sc_gather(W, U, H, I):
    info = plsc.get_sparse_core_info()
    nc, ns = info.num_cores, info.num_subcores
    nw = nc * ns
    bpw = _B // nw
    mesh = plsc.VectorSubcoreMesh(core_axis_name="c", subcore_axis_name="s")

    @functools.partial(
        pl.kernel,
        mesh=mesh,
        compiler_params=pltpu.CompilerParams(needs_layout_passes=False),
        out_type=[
            jax.ShapeDtypeStruct((_B, _D), jnp.float32),
            jax.ShapeDtypeStruct((_B, _D), jnp.float32),
        ],
        scratch_types=[
            pltpu.VMEM((bpw,), jnp.int32),
            pltpu.VMEM((bpw, _D), jnp.float32),
            pltpu.SemaphoreType.DMA,
        ],
    )
    def gather_k(W_hbm, U_hbm, H_hbm, I_hbm, wout, hout, vidx, rows, sem):
        wid = lax.axis_index("s") * nc + lax.axis_index("c")
        base = wid * bpw
        iota16 = lax.iota(jnp.int32, 16)

        for idx_hbm, tab_hbm, out_hbm in ((U_hbm, W_hbm, wout),
                                          (I_hbm, H_hbm, hout)):
            pltpu.sync_copy(idx_hbm.at[pl.ds(base, bpw)], vidx)
            for g in range(bpw // 16):
                sv = vidx[pl.ds(16 * g, 16)]
                for lane in range(16):
                    j = 16 * g + lane
                    u = jnp.sum(jnp.where(iota16 == lane, sv, 0))
                    pltpu.make_async_copy(
                        tab_hbm.at[pl.ds(u, 1)], rows.at[pl.ds(j, 1)], sem
                    ).start()
            pltpu.make_async_copy(
                tab_hbm.at[pl.ds(0, bpw)], rows, sem
            ).wait()
            pltpu.sync_copy(rows, out_hbm.at[pl.ds(base, bpw)])

    return gather_k(W, U, H, I)


def _tc_matmul(wu, hi):
    bm = 512

    def mm(w_ref, h_ref, o_ref):
        o_ref[...] = lax.dot_general(
            w_ref[...].astype(jnp.bfloat16), h_ref[...].astype(jnp.bfloat16),
            (((1,), (1,)), ((), ())),
            preferred_element_type=jnp.float32,
        )

    return pl.pallas_call(
        mm,
        grid=(_B // bm,),
        in_specs=[
            pl.BlockSpec((bm, _D), lambda i: (i, 0)),
            pl.BlockSpec((_B, _D), lambda i: (0, 0)),
        ],
        out_specs=pl.BlockSpec((bm, _B), lambda i: (i, 0)),
        out_shape=jax.ShapeDtypeStruct((_B, _B), jnp.float32),
    )(wu, hi)


def kernel(U, I, W, H):
    wu, hi = _sc_gather(W, U, H, I)
    return _tc_matmul(wu, hi)
